# baseline (speedup 1.0000x reference)
import functools

import jax
import jax.numpy as jnp
from jax import lax
from jax.experimental import pallas as pl
from jax.experimental.pallas import tpu as pltpu
from jax.experimental.pallas import tpu_sc as plsc

NC = 2
NS = 16
CH = 128
F2 = 64
NBUF = 2


def _sc_mesh():
    return plsc.VectorSubcoreMesh(
        core_axis_name="c", subcore_axis_name="s", num_cores=NC, num_subcores=NS
    )


def _make_deg(nch, n_acc):
    hch = nch // NC

    @functools.partial(
        pl.kernel,
        mesh=_sc_mesh(),
        out_type=jax.ShapeDtypeStruct((NC, n_acc), jnp.float32),
        scratch_types=[
            pltpu.VMEM((hch, CH), jnp.int32),
            pltpu.VMEM((CH,), jnp.float32),
            pltpu.VMEM_SHARED((n_acc,), jnp.float32),
        ],
    )
    def deg_kernel(dst_hbm, zeros_hbm, out_hbm, dst_v, ones_v, acc_sh):
        c = lax.axis_index("c")
        s = lax.axis_index("s")
        span = n_acc // NS
        pltpu.sync_copy(zeros_hbm, acc_sh.at[pl.ds(s * span, span)])
        for k in range(CH // 16):
            ones_v[pl.ds(k * 16, 16)] = jnp.full((16,), 1.0, jnp.float32)
        pltpu.sync_copy(dst_hbm.at[s, c], dst_v)
        plsc.subcore_barrier()

        def body(j, carry):
            pltpu.sync_copy(ones_v, acc_sh.at[dst_v.at[j]], add=True)
            return carry

        lax.fori_loop(0, hch, body, 0)
        plsc.subcore_barrier()
        pltpu.sync_copy(
            acc_sh.at[pl.ds(s * span, span)], out_hbm.at[c].at[pl.ds(s * span, span)]
        )

    return deg_kernel


def _make_segsum(nch, n_acc, n_tab):
    zspan = n_acc // NS

    @functools.partial(
        pl.kernel,
        mesh=_sc_mesh(),
        out_type=jax.ShapeDtypeStruct((NC, n_acc, F2), jnp.float32),
        scratch_types=[
            pltpu.VMEM((nch // 2, CH), jnp.int32),
            pltpu.VMEM((nch // 2, CH), jnp.int32),
            [pltpu.VMEM((CH, F2), jnp.float32)] * NBUF,
            pltpu.VMEM_SHARED((n_acc, F2), jnp.float32),
            pltpu.VMEM_SHARED((n_tab, F2), jnp.float32),
            [pltpu.SemaphoreType.DMA] * NBUF,
            [pltpu.SemaphoreType.DMA] * NBUF,
        ],
        compiler_params=pltpu.CompilerParams(use_tc_tiling_on_sc=False),
    )
    def segsum(src_hbm, dst_hbm, g_hbm, zeros_hbm, out_hbm,
               src_v, dst_v, rows, acc_sh, tab_sh, semg, sems):
        c = lax.axis_index("c")
        s = lax.axis_index("s")
        hch = nch // 2
        pltpu.sync_copy(zeros_hbm, acc_sh.at[pl.ds(s * zspan, zspan)])
        tspan = n_tab // NS
        pltpu.sync_copy(
            g_hbm.at[pl.ds(c * n_tab + s * tspan, tspan)],
            tab_sh.at[pl.ds(s * tspan, tspan)],
        )
        plsc.subcore_barrier()

        hbm_tab = g_hbm.at[pl.ds(c * n_tab, n_tab)]
        for h in range(2):
            pltpu.sync_copy(src_hbm.at[s].at[pl.ds(h * hch, hch)], src_v)
            pltpu.sync_copy(dst_hbm.at[s].at[pl.ds(h * hch, hch)], dst_v)
            pltpu.async_copy(tab_sh.at[src_v.at[0]], rows[0], semg[0])

            def body(i, carry):
                j = 2 * i
                pltpu.async_copy(hbm_tab.at[src_v.at[j + 1]], rows[1], semg[1])
                pltpu.make_async_copy(tab_sh.at[src_v.at[j]], rows[0], semg[0]).wait()
                pltpu.sync_copy(rows[0], acc_sh.at[dst_v.at[j]], add=True)

                @pl.when(j + 2 < hch)
                def _():
                    pltpu.async_copy(tab_sh.at[src_v.at[j + 2]], rows[0], semg[0])

                pltpu.make_async_copy(hbm_tab.at[src_v.at[j + 1]], rows[1], semg[1]).wait()
                pltpu.sync_copy(rows[1], acc_sh.at[dst_v.at[j + 1]], add=True)
                return carry

            lax.fori_loop(0, hch // 2, body, 0)
        plsc.subcore_barrier()
        pltpu.sync_copy(
            acc_sh.at[pl.ds(s * zspan, zspan)],
            out_hbm.at[c].at[pl.ds(s * zspan, zspan)],
        )

    return segsum


def _tc_g1(x_ref, w_ref, deg_ref, out_ref):
    dinv = lax.rsqrt(1.0 + deg_ref[:, 0:1] + deg_ref[:, 1:2])
    h = jnp.dot(x_ref[...], w_ref[...], preferred_element_type=jnp.float32)
    g = h * dinv
    out_ref[0] = g[:, 0:F2]
    out_ref[1] = g[:, F2:]


def _relu_in(s_ref, g_ref, deg_ref, b_ref):
    dinv = lax.rsqrt(1.0 + deg_ref[:, 0:1] + deg_ref[:, 1:2])
    t = jnp.concatenate(
        [s_ref[0] + g_ref[0], s_ref[1] + g_ref[1]], axis=1
    ) * dinv + b_ref[...]
    return jnp.maximum(t, 0.0), dinv


def _tc_mid(s_ref, g_ref, deg_ref, b_ref, w_ref, out_ref):
    z, dinv = _relu_in(s_ref, g_ref, deg_ref, b_ref)
    g = jnp.dot(z, w_ref[...], preferred_element_type=jnp.float32) * dinv
    out_ref[0] = g[:, 0:F2]
    out_ref[1] = g[:, F2:]


def _lse(a):
    m = jnp.max(a, axis=1, keepdims=True)
    return m + jnp.log(jnp.sum(jnp.exp(a - m), axis=1, keepdims=True))


def _tc_head(s_ref, g_ref, deg_ref, b_ref, wh_ref, bh_ref, out_ref):
    z, _ = _relu_in(s_ref, g_ref, deg_ref, b_ref)
    logits = jnp.dot(z, wh_ref[...], preferred_element_type=jnp.float32) + bh_ref[...]
    det = logits[:, 0:2]
    u = logits[:, 2:4]
    r = logits[:, 4:8]
    det_lp = det - _lse(det)
    u_lp = u - _lse(u)
    r_lp = r - _lse(r)
    mask = det[:, 1:2] > det[:, 0:1]
    neg_inf = jnp.full_like(u_lp[:, 0:1], -jnp.inf)
    root0 = jnp.where(mask, neg_inf, u_lp[:, 0:1])
    root1 = jnp.where(mask, r_lp[:, 0:1], u_lp[:, 1:2])
    root234 = jnp.where(mask, r_lp[:, 1:4], neg_inf)
    pad = jnp.zeros_like(u_lp[:, 0:1])
    out_ref[...] = jnp.concatenate([det_lp, root0, root1, root234, pad], axis=1)


def kernel(x, edge_index, W1, b1, W2, b2, Wd, bd, Wu, bu, Wr, br):
    n, f = x.shape
    e = edge_index.shape[1]
    src = edge_index[0].astype(jnp.int32)
    dst = edge_index[1].astype(jnp.int32)

    unit = 4 * CH
    ept = ((e + NS * unit - 1) // (NS * unit)) * unit
    nch = ept // CH
    pad = ept * NS - e
    srcp = jnp.concatenate([src, jnp.zeros((pad,), jnp.int32)]).reshape(NS, nch, CH)
    dstp = jnp.concatenate([dst, jnp.full((pad,), n, jnp.int32)]).reshape(NS, nch, CH)

    n_acc = ((n // NS + 8) // 8) * 8 * NS
    n_deg = ((n // (NS * CH)) + 1) * NS * CH
    zeros_seg = jnp.zeros((n_acc // NS, F2), jnp.float32)
    zeros_deg = jnp.zeros((n_deg // NS,), jnp.float32)

    dstp_deg = dstp.reshape(NS, NC, nch // NC, CH)
    degp = _make_deg(nch, n_deg)(dstp_deg, zeros_deg)
    degT = jnp.transpose(degp[:, :n])

    segsum = _make_segsum(nch, n_acc, n)

    RB = 2000
    grid = (n // RB,)
    row_spec = pl.BlockSpec((RB, f), lambda i: (i, 0))
    half_spec = pl.BlockSpec((NC, RB, F2), lambda i: (0, i, 0))
    deg_spec = pl.BlockSpec((RB, NC), lambda i: (i, 0))
    w_spec = pl.BlockSpec((f, f), lambda i: (0, 0))
    b_spec = pl.BlockSpec((f,), lambda i: (0,))
    halves_shape = jax.ShapeDtypeStruct((NC, n, F2), jnp.float32)

    g1 = pl.pallas_call(
        _tc_g1,
        grid=grid,
        in_specs=[row_spec, w_spec, deg_spec],
        out_specs=half_spec,
        out_shape=halves_shape,
    )(x, W1, degT)
    S1 = segsum(srcp, dstp, g1.reshape(NC * n, F2), zeros_seg)
    g2 = pl.pallas_call(
        _tc_mid,
        grid=grid,
        in_specs=[half_spec, half_spec, deg_spec, b_spec, w_spec],
        out_specs=half_spec,
        out_shape=halves_shape,
    )(S1, g1, degT, b1, W2)
    S2 = segsum(srcp, dstp, g2.reshape(NC * n, F2), zeros_seg)

    Wh = jnp.concatenate([Wd, Wu, Wr], axis=1)
    bh = jnp.concatenate([bd, bu, br])
    out8 = pl.pallas_call(
        _tc_head,
        grid=grid,
        in_specs=[half_spec, half_spec, deg_spec, b_spec,
                  pl.BlockSpec((f, 8), lambda i: (0, 0)),
                  pl.BlockSpec((8,), lambda i: (0,))],
        out_specs=pl.BlockSpec((RB, 8), lambda i: (i, 0)),
        out_shape=jax.ShapeDtypeStruct((n, 8), jnp.float32),
    )(S2, g2, degT, b2, Wh, bh)
    return out8[:, 0:2], out8[:, 2:7]


# speedup vs baseline: 1.3843x; 1.3843x over previous
import functools

import jax
import jax.numpy as jnp
from jax import lax
from jax.experimental import pallas as pl
from jax.experimental.pallas import tpu as pltpu
from jax.experimental.pallas import tpu_sc as plsc

NC = 2
NS = 16
CH = 128
F2 = 64
NBUF = 2


def _sc_mesh():
    return plsc.VectorSubcoreMesh(
        core_axis_name="c", subcore_axis_name="s", num_cores=NC, num_subcores=NS
    )


def _make_deg(nch, n_acc):
    hch = nch // NC

    @functools.partial(
        pl.kernel,
        mesh=_sc_mesh(),
        out_type=jax.ShapeDtypeStruct((NC, n_acc), jnp.float32),
        scratch_types=[
            pltpu.VMEM((hch, CH), jnp.int32),
            pltpu.VMEM((CH,), jnp.float32),
            pltpu.VMEM_SHARED((n_acc,), jnp.float32),
            pltpu.SemaphoreType.DMA,
        ],
    )
    def deg_kernel(dst_hbm, zeros_hbm, out_hbm, dst_v, ones_v, acc_sh, sem):
        c = lax.axis_index("c")
        s = lax.axis_index("s")
        span = n_acc // NS
        pltpu.sync_copy(zeros_hbm, acc_sh.at[pl.ds(s * span, span)])
        for k in range(CH // 16):
            ones_v[pl.ds(k * 16, 16)] = jnp.full((16,), 1.0, jnp.float32)
        pltpu.sync_copy(dst_hbm.at[s, c], dst_v)
        plsc.subcore_barrier()

        def body(i, carry):
            j = 8 * i
            for b in range(8):
                pltpu.async_copy(ones_v, acc_sh.at[dst_v.at[j + b]], sem, add=True)
            for b in range(8):
                pltpu.make_async_copy(ones_v, acc_sh.at[dst_v.at[0]], sem).wait()
            return carry

        lax.fori_loop(0, hch // 8, body, 0)
        plsc.subcore_barrier()
        pltpu.sync_copy(
            acc_sh.at[pl.ds(s * span, span)], out_hbm.at[c].at[pl.ds(s * span, span)]
        )

    return deg_kernel


def _make_segsum(nch, n_acc, n_tab):
    zspan = n_acc // NS

    @functools.partial(
        pl.kernel,
        mesh=_sc_mesh(),
        out_type=jax.ShapeDtypeStruct((NC, n_acc, F2), jnp.float32),
        scratch_types=[
            pltpu.VMEM((nch // 2, CH), jnp.int32),
            pltpu.VMEM((nch // 2, CH), jnp.int32),
            [pltpu.VMEM((CH, F2), jnp.float32)] * NBUF,
            pltpu.VMEM_SHARED((n_acc, F2), jnp.float32),
            pltpu.VMEM_SHARED((n_tab, F2), jnp.float32),
            [pltpu.SemaphoreType.DMA] * NBUF,
            [pltpu.SemaphoreType.DMA] * NBUF,
        ],
        compiler_params=pltpu.CompilerParams(use_tc_tiling_on_sc=False),
    )
    def segsum(src_hbm, dst_hbm, g_hbm, zeros_hbm, out_hbm,
               src_v, dst_v, rows, acc_sh, tab_sh, semg, sems):
        c = lax.axis_index("c")
        s = lax.axis_index("s")
        hch = nch // 2
        pltpu.sync_copy(zeros_hbm, acc_sh.at[pl.ds(s * zspan, zspan)])
        tspan = n_tab // NS
        pltpu.sync_copy(
            g_hbm.at[pl.ds(c * n_tab + s * tspan, tspan)],
            tab_sh.at[pl.ds(s * tspan, tspan)],
        )
        plsc.subcore_barrier()

        for h in range(2):
            pltpu.sync_copy(src_hbm.at[s].at[pl.ds(h * hch, hch)], src_v)
            pltpu.sync_copy(dst_hbm.at[s].at[pl.ds(h * hch, hch)], dst_v)
            pltpu.async_copy(tab_sh.at[src_v.at[0]], rows[0], semg[0])

            def body(i, carry):
                j = 2 * i
                pltpu.async_copy(tab_sh.at[src_v.at[j + 1]], rows[1], semg[1])
                pltpu.make_async_copy(tab_sh.at[src_v.at[j]], rows[0], semg[0]).wait()
                pltpu.sync_copy(rows[0], acc_sh.at[dst_v.at[j]], add=True)

                @pl.when(j + 2 < hch)
                def _():
                    pltpu.async_copy(tab_sh.at[src_v.at[j + 2]], rows[0], semg[0])

                pltpu.make_async_copy(tab_sh.at[src_v.at[j + 1]], rows[1], semg[1]).wait()
                pltpu.sync_copy(rows[1], acc_sh.at[dst_v.at[j + 1]], add=True)
                return carry

            lax.fori_loop(0, hch // 2, body, 0)
        plsc.subcore_barrier()
        pltpu.sync_copy(
            acc_sh.at[pl.ds(s * zspan, zspan)],
            out_hbm.at[c].at[pl.ds(s * zspan, zspan)],
        )

    return segsum


def _tc_g1(x_ref, w_ref, deg_ref, out_ref):
    dinv = lax.rsqrt(1.0 + deg_ref[:, 0:1] + deg_ref[:, 1:2])
    h = jnp.dot(x_ref[...], w_ref[...], preferred_element_type=jnp.float32)
    g = h * dinv
    out_ref[0] = g[:, 0:F2]
    out_ref[1] = g[:, F2:]


def _relu_in(s_ref, g_ref, deg_ref, b_ref):
    dinv = lax.rsqrt(1.0 + deg_ref[:, 0:1] + deg_ref[:, 1:2])
    t = jnp.concatenate(
        [s_ref[0] + g_ref[0], s_ref[1] + g_ref[1]], axis=1
    ) * dinv + b_ref[...]
    return jnp.maximum(t, 0.0), dinv


def _tc_mid(s_ref, g_ref, deg_ref, b_ref, w_ref, out_ref):
    z, dinv = _relu_in(s_ref, g_ref, deg_ref, b_ref)
    g = jnp.dot(z, w_ref[...], preferred_element_type=jnp.float32) * dinv
    out_ref[0] = g[:, 0:F2]
    out_ref[1] = g[:, F2:]


def _lse(a):
    m = jnp.max(a, axis=1, keepdims=True)
    return m + jnp.log(jnp.sum(jnp.exp(a - m), axis=1, keepdims=True))


def _tc_head(s_ref, g_ref, deg_ref, b_ref, wh_ref, bh_ref, det_ref, root_ref):
    z, _ = _relu_in(s_ref, g_ref, deg_ref, b_ref)
    logits = jnp.dot(z, wh_ref[...], preferred_element_type=jnp.float32) + bh_ref[...]
    det = logits[:, 0:2]
    u = logits[:, 2:4]
    r = logits[:, 4:8]
    u_lp = u - _lse(u)
    r_lp = r - _lse(r)
    mask = det[:, 1:2] > det[:, 0:1]
    neg_inf = jnp.full_like(u_lp[:, 0:1], -jnp.inf)
    root0 = jnp.where(mask, neg_inf, u_lp[:, 0:1])
    root1 = jnp.where(mask, r_lp[:, 0:1], u_lp[:, 1:2])
    root234 = jnp.where(mask, r_lp[:, 1:4], neg_inf)
    det_ref[...] = det - _lse(det)
    root_ref[...] = jnp.concatenate([root0, root1, root234], axis=1)


def kernel(x, edge_index, W1, b1, W2, b2, Wd, bd, Wu, bu, Wr, br):
    n, f = x.shape
    e = edge_index.shape[1]
    src = edge_index[0].astype(jnp.int32)
    dst = edge_index[1].astype(jnp.int32)

    unit = 16 * CH
    ept = ((e + NS * unit - 1) // (NS * unit)) * unit
    nch = ept // CH
    pad = ept * NS - e
    srcp = jnp.concatenate([src, jnp.zeros((pad,), jnp.int32)]).reshape(NS, nch, CH)
    dstp = jnp.concatenate([dst, jnp.full((pad,), n, jnp.int32)]).reshape(NS, nch, CH)

    n_acc = ((n // NS + 8) // 8) * 8 * NS
    n_deg = ((n // (NS * CH)) + 1) * NS * CH
    zeros_seg = jnp.zeros((n_acc // NS, F2), jnp.float32)
    zeros_deg = jnp.zeros((n_deg // NS,), jnp.float32)

    dstp_deg = dstp.reshape(NS, NC, nch // NC, CH)
    degp = _make_deg(nch, n_deg)(dstp_deg, zeros_deg)
    degT = jnp.transpose(degp[:, :n])

    segsum = _make_segsum(nch, n_acc, n)

    RB = 2000
    grid = (n // RB,)
    row_spec = pl.BlockSpec((RB, f), lambda i: (i, 0))
    half_spec = pl.BlockSpec((NC, RB, F2), lambda i: (0, i, 0))
    deg_spec = pl.BlockSpec((RB, NC), lambda i: (i, 0))
    w_spec = pl.BlockSpec((f, f), lambda i: (0, 0))
    b_spec = pl.BlockSpec((f,), lambda i: (0,))
    halves_shape = jax.ShapeDtypeStruct((NC, n, F2), jnp.float32)

    g1 = pl.pallas_call(
        _tc_g1,
        grid=grid,
        in_specs=[row_spec, w_spec, deg_spec],
        out_specs=half_spec,
        out_shape=halves_shape,
    )(x, W1, degT)
    S1 = segsum(srcp, dstp, g1.reshape(NC * n, F2), zeros_seg)
    g2 = pl.pallas_call(
        _tc_mid,
        grid=grid,
        in_specs=[half_spec, half_spec, deg_spec, b_spec, w_spec],
        out_specs=half_spec,
        out_shape=halves_shape,
    )(S1, g1, degT, b1, W2)
    S2 = segsum(srcp, dstp, g2.reshape(NC * n, F2), zeros_seg)

    Wh = jnp.concatenate([Wd, Wu, Wr], axis=1)
    bh = jnp.concatenate([bd, bu, br])
    det_lp, root_lp = pl.pallas_call(
        _tc_head,
        grid=grid,
        in_specs=[half_spec, half_spec, deg_spec, b_spec,
                  pl.BlockSpec((f, 8), lambda i: (0, 0)),
                  pl.BlockSpec((8,), lambda i: (0,))],
        out_specs=[pl.BlockSpec((RB, 2), lambda i: (i, 0)),
                   pl.BlockSpec((RB, 5), lambda i: (i, 0))],
        out_shape=[jax.ShapeDtypeStruct((n, 2), jnp.float32),
                   jax.ShapeDtypeStruct((n, 5), jnp.float32)],
    )(S2, g2, degT, b2, Wh, bh)
    return det_lp, root_lp


# speedup vs baseline: 1.3861x; 1.0013x over previous
import functools

import jax
import jax.numpy as jnp
from jax import lax
from jax.experimental import pallas as pl
from jax.experimental.pallas import tpu as pltpu
from jax.experimental.pallas import tpu_sc as plsc

NC = 2
NS = 16
CH = 128
F2 = 64
NBUF = 2


def _sc_mesh():
    return plsc.VectorSubcoreMesh(
        core_axis_name="c", subcore_axis_name="s", num_cores=NC, num_subcores=NS
    )


def _make_deg(nch, n_acc):
    hch = nch // NC

    @functools.partial(
        pl.kernel,
        mesh=_sc_mesh(),
        out_type=jax.ShapeDtypeStruct((NC, n_acc), jnp.float32),
        scratch_types=[
            pltpu.VMEM((hch, CH), jnp.int32),
            pltpu.VMEM((CH,), jnp.float32),
            pltpu.VMEM_SHARED((n_acc,), jnp.float32),
            pltpu.SemaphoreType.DMA,
        ],
    )
    def deg_kernel(dst_hbm, zeros_hbm, out_hbm, dst_v, ones_v, acc_sh, sem):
        c = lax.axis_index("c")
        s = lax.axis_index("s")
        span = n_acc // NS
        pltpu.sync_copy(zeros_hbm, acc_sh.at[pl.ds(s * span, span)])
        for k in range(CH // 16):
            ones_v[pl.ds(k * 16, 16)] = jnp.full((16,), 1.0, jnp.float32)
        pltpu.sync_copy(dst_hbm.at[s, c], dst_v)
        plsc.subcore_barrier()

        def body(i, carry):
            j = 8 * i
            for b in range(8):
                pltpu.async_copy(ones_v, acc_sh.at[dst_v.at[j + b]], sem, add=True)
            for b in range(8):
                pltpu.make_async_copy(ones_v, acc_sh.at[dst_v.at[0]], sem).wait()
            return carry

        lax.fori_loop(0, hch // 8, body, 0)
        plsc.subcore_barrier()
        pltpu.sync_copy(
            acc_sh.at[pl.ds(s * span, span)], out_hbm.at[c].at[pl.ds(s * span, span)]
        )

    return deg_kernel


def _make_segsum(nch, n_acc, n_tab):
    zspan = n_acc // NS

    @functools.partial(
        pl.kernel,
        mesh=_sc_mesh(),
        out_type=jax.ShapeDtypeStruct((NC, n_acc, F2), jnp.float32),
        scratch_types=[
            pltpu.VMEM((nch // 2, CH), jnp.int32),
            pltpu.VMEM((nch // 2, CH), jnp.int32),
            [pltpu.VMEM((CH, F2), jnp.float32)] * NBUF,
            pltpu.VMEM_SHARED((n_acc, F2), jnp.float32),
            pltpu.VMEM_SHARED((n_tab, F2), jnp.float32),
            [pltpu.SemaphoreType.DMA] * NBUF,
        ],
        compiler_params=pltpu.CompilerParams(use_tc_tiling_on_sc=False),
    )
    def segsum(src_hbm, dst_hbm, g_hbm, zeros_hbm, out_hbm,
               src_v, dst_v, rows, acc_sh, tab_sh, semg):
        c = lax.axis_index("c")
        s = lax.axis_index("s")
        hch = nch // 2
        pltpu.sync_copy(zeros_hbm, acc_sh.at[pl.ds(s * zspan, zspan)])
        tspan = n_tab // NS
        pltpu.sync_copy(
            g_hbm.at[pl.ds(c * n_tab + s * tspan, tspan)],
            tab_sh.at[pl.ds(s * tspan, tspan)],
        )
        plsc.subcore_barrier()

        for h in range(2):
            pltpu.sync_copy(src_hbm.at[s].at[pl.ds(h * hch, hch)], src_v)
            pltpu.sync_copy(dst_hbm.at[s].at[pl.ds(h * hch, hch)], dst_v)
            pltpu.async_copy(tab_sh.at[src_v.at[0]], rows[0], semg[0])

            def body(i, carry):
                j = 2 * i
                pltpu.async_copy(tab_sh.at[src_v.at[j + 1]], rows[1], semg[1])
                pltpu.make_async_copy(tab_sh.at[src_v.at[j]], rows[0], semg[0]).wait()
                pltpu.sync_copy(rows[0], acc_sh.at[dst_v.at[j]], add=True)

                @pl.when(j + 2 < hch)
                def _():
                    pltpu.async_copy(tab_sh.at[src_v.at[j + 2]], rows[0], semg[0])

                pltpu.make_async_copy(tab_sh.at[src_v.at[j + 1]], rows[1], semg[1]).wait()
                pltpu.sync_copy(rows[1], acc_sh.at[dst_v.at[j + 1]], add=True)
                return carry

            lax.fori_loop(0, hch // 2, body, 0)
        plsc.subcore_barrier()
        pltpu.sync_copy(
            acc_sh.at[pl.ds(s * zspan, zspan)],
            out_hbm.at[c].at[pl.ds(s * zspan, zspan)],
        )

    return segsum


def _tc_g1(x_ref, w_ref, deg_ref, out_ref):
    dinv = lax.rsqrt(1.0 + deg_ref[:, 0:1] + deg_ref[:, 1:2])
    h = jnp.dot(x_ref[...], w_ref[...], preferred_element_type=jnp.float32)
    g = h * dinv
    out_ref[0] = g[:, 0:F2]
    out_ref[1] = g[:, F2:]


def _relu_in(s_ref, g_ref, deg_ref, b_ref):
    dinv = lax.rsqrt(1.0 + deg_ref[:, 0:1] + deg_ref[:, 1:2])
    t = jnp.concatenate(
        [s_ref[0] + g_ref[0], s_ref[1] + g_ref[1]], axis=1
    ) * dinv + b_ref[...]
    return jnp.maximum(t, 0.0), dinv


def _tc_mid(s_ref, g_ref, deg_ref, b_ref, w_ref, out_ref):
    z, dinv = _relu_in(s_ref, g_ref, deg_ref, b_ref)
    g = jnp.dot(z, w_ref[...], preferred_element_type=jnp.float32) * dinv
    out_ref[0] = g[:, 0:F2]
    out_ref[1] = g[:, F2:]


def _lse(a):
    m = jnp.max(a, axis=1, keepdims=True)
    return m + jnp.log(jnp.sum(jnp.exp(a - m), axis=1, keepdims=True))


def _tc_head(s_ref, g_ref, deg_ref, b_ref, wh_ref, bh_ref, det_ref, root_ref):
    z, _ = _relu_in(s_ref, g_ref, deg_ref, b_ref)
    logits = jnp.dot(z, wh_ref[...], preferred_element_type=jnp.float32) + bh_ref[...]
    det = logits[:, 0:2]
    u = logits[:, 2:4]
    r = logits[:, 4:8]
    u_lp = u - _lse(u)
    r_lp = r - _lse(r)
    mask = det[:, 1:2] > det[:, 0:1]
    neg_inf = jnp.full_like(u_lp[:, 0:1], -jnp.inf)
    root0 = jnp.where(mask, neg_inf, u_lp[:, 0:1])
    root1 = jnp.where(mask, r_lp[:, 0:1], u_lp[:, 1:2])
    root234 = jnp.where(mask, r_lp[:, 1:4], neg_inf)
    det_ref[...] = det - _lse(det)
    root_ref[...] = jnp.concatenate([root0, root1, root234], axis=1)


def kernel(x, edge_index, W1, b1, W2, b2, Wd, bd, Wu, bu, Wr, br):
    n, f = x.shape
    e = edge_index.shape[1]
    src = edge_index[0].astype(jnp.int32)
    dst = edge_index[1].astype(jnp.int32)

    unit = 16 * CH
    ept = ((e + NS * unit - 1) // (NS * unit)) * unit
    nch = ept // CH
    pad = ept * NS - e
    srcp = jnp.concatenate([src, jnp.zeros((pad,), jnp.int32)]).reshape(NS, nch, CH)
    dstp = jnp.concatenate([dst, jnp.full((pad,), n, jnp.int32)]).reshape(NS, nch, CH)

    n_acc = ((n // NS + 8) // 8) * 8 * NS
    n_deg = ((n // (NS * CH)) + 1) * NS * CH
    zeros_seg = jnp.zeros((n_acc // NS, F2), jnp.float32)
    zeros_deg = jnp.zeros((n_deg // NS,), jnp.float32)

    dstp_deg = dstp.reshape(NS, NC, nch // NC, CH)
    degp = _make_deg(nch, n_deg)(dstp_deg, zeros_deg)
    degT = jnp.transpose(degp[:, :n])

    segsum = _make_segsum(nch, n_acc, n)

    RB = 2000
    grid = (n // RB,)
    row_spec = pl.BlockSpec((RB, f), lambda i: (i, 0))
    half_spec = pl.BlockSpec((NC, RB, F2), lambda i: (0, i, 0))
    deg_spec = pl.BlockSpec((RB, NC), lambda i: (i, 0))
    w_spec = pl.BlockSpec((f, f), lambda i: (0, 0))
    b_spec = pl.BlockSpec((f,), lambda i: (0,))
    halves_shape = jax.ShapeDtypeStruct((NC, n, F2), jnp.float32)

    g1 = pl.pallas_call(
        _tc_g1,
        grid=grid,
        in_specs=[row_spec, w_spec, deg_spec],
        out_specs=half_spec,
        out_shape=halves_shape,
    )(x, W1, degT)
    S1 = segsum(srcp, dstp, g1.reshape(NC * n, F2), zeros_seg)
    g2 = pl.pallas_call(
        _tc_mid,
        grid=grid,
        in_specs=[half_spec, half_spec, deg_spec, b_spec, w_spec],
        out_specs=half_spec,
        out_shape=halves_shape,
    )(S1, g1, degT, b1, W2)
    S2 = segsum(srcp, dstp, g2.reshape(NC * n, F2), zeros_seg)

    Wh = jnp.concatenate([Wd, Wu, Wr], axis=1)
    bh = jnp.concatenate([bd, bu, br])
    det_lp, root_lp = pl.pallas_call(
        _tc_head,
        grid=grid,
        in_specs=[half_spec, half_spec, deg_spec, b_spec,
                  pl.BlockSpec((f, 8), lambda i: (0, 0)),
                  pl.BlockSpec((8,), lambda i: (0,))],
        out_specs=[pl.BlockSpec((RB, 2), lambda i: (i, 0)),
                   pl.BlockSpec((RB, 5), lambda i: (i, 0))],
        out_shape=[jax.ShapeDtypeStruct((n, 2), jnp.float32),
                   jax.ShapeDtypeStruct((n, 5), jnp.float32)],
    )(S2, g2, degT, b2, Wh, bh)
    return det_lp, root_lp
